# baseline (speedup 1.0000x reference)
import jax
import jax.numpy as jnp
from jax import lax
from jax.experimental import pallas as pl
from jax.experimental.pallas import tpu as pltpu
from jax.experimental.pallas import tpu_sc as plsc

VOCAB = 100000
D = 128
B = 4096
N = 50
NPAD = 56

NC = 2
NS = 16
NW = NC * NS

B_PER_W = B // NW
NBUF = 8
ROUNDS = B_PER_W // NBUF


def _body(table_hbm, idx_hbm, out_hbm, idx_v, rows_v, gsem, wsem):
    wid = lax.axis_index("s") * NC + lax.axis_index("c")
    b0 = wid * B_PER_W
    pltpu.sync_copy(idx_hbm.at[wid], idx_v)

    def gather(c, b):
        pltpu.async_copy(table_hbm.at[idx_v.at[c]], rows_v.at[b], gsem.at[b])

    def wait_gather(b):
        pltpu.make_async_copy(
            table_hbm.at[idx_v.at[0]], rows_v.at[b], gsem.at[b]).wait()

    def write(c, b):
        pltpu.async_copy(
            rows_v.at[b, pl.ds(0, N)], out_hbm.at[b0 + c], wsem.at[b])

    def wait_write(b):
        pltpu.make_async_copy(
            rows_v.at[b, pl.ds(0, N)], out_hbm.at[b0], wsem.at[b]).wait()

    for b in range(NBUF):
        gather(b, b)

    def round_body(r, carry):
        c0 = r * NBUF
        for b in range(NBUF):
            wait_gather(b)
            write(c0 + b, b)
        for b in range(NBUF):
            wait_write(b)
            gather(c0 + NBUF + b, b)
        return carry

    lax.fori_loop(0, ROUNDS - 1, round_body, 0)

    c0 = (ROUNDS - 1) * NBUF
    for b in range(NBUF):
        wait_gather(b)
        write(c0 + b, b)
    for b in range(NBUF):
        wait_write(b)


@jax.jit
def kernel(x, table):
    idx = jnp.pad(x.astype(jnp.int32), ((0, 0), (0, NPAD - N)))
    idx = idx.reshape(NW, B_PER_W, NPAD)
    mesh = plsc.VectorSubcoreMesh(core_axis_name="c", subcore_axis_name="s")
    return pl.kernel(
        _body,
        out_type=jax.ShapeDtypeStruct((B, N, D), jnp.float32),
        mesh=mesh,
        scratch_types=[
            pltpu.VMEM((B_PER_W, NPAD), jnp.int32),
            pltpu.VMEM((NBUF, NPAD, D), jnp.float32),
            pltpu.SemaphoreType.DMA((NBUF,)),
            pltpu.SemaphoreType.DMA((NBUF,)),
        ],
    )(table, idx)


# speedup vs baseline: 13.1005x; 13.1005x over previous
import jax
import jax.numpy as jnp
from jax import lax
from jax.experimental import pallas as pl
from jax.experimental.pallas import tpu as pltpu
from jax.experimental.pallas import tpu_sc as plsc

VOCAB = 100000
D = 128
B = 4096
N = 50

NC = 2
NS = 16
NW = NC * NS

CHUNK = B // NW
NBUF = 5
ROUNDS = N // NBUF


def _body(table_hbm, idx_hbm, out_hbm, idx_v, rows_v, gsem, wsem):
    wid = lax.axis_index("s") * NC + lax.axis_index("c")
    b0 = wid * CHUNK
    pltpu.sync_copy(idx_hbm.at[:, pl.ds(b0, CHUNK)], idx_v)

    def gather(n, b):
        pltpu.async_copy(table_hbm.at[idx_v.at[n]], rows_v.at[b], gsem.at[b])

    def wait_gather(b):
        pltpu.make_async_copy(
            table_hbm.at[idx_v.at[0]], rows_v.at[b], gsem.at[b]).wait()

    def write(n, b):
        pltpu.async_copy(
            rows_v.at[b], out_hbm.at[n, pl.ds(b0, CHUNK)], wsem.at[b])

    def wait_write(b):
        pltpu.make_async_copy(
            rows_v.at[b], out_hbm.at[0, pl.ds(b0, CHUNK)], wsem.at[b]).wait()

    for b in range(NBUF):
        gather(b, b)

    def round_body(r, carry):
        n0 = r * NBUF
        for b in range(NBUF):
            wait_gather(b)
            write(n0 + b, b)
        for b in range(NBUF):
            wait_write(b)
            gather(n0 + NBUF + b, b)
        return carry

    lax.fori_loop(0, ROUNDS - 1, round_body, 0)

    n0 = (ROUNDS - 1) * NBUF
    for b in range(NBUF):
        wait_gather(b)
        write(n0 + b, b)
    for b in range(NBUF):
        wait_write(b)


@jax.jit
def kernel(x, table):
    idx = x.astype(jnp.int32).T
    mesh = plsc.VectorSubcoreMesh(core_axis_name="c", subcore_axis_name="s")
    out_t = pl.kernel(
        _body,
        out_type=jax.ShapeDtypeStruct((N, B, D), jnp.float32),
        mesh=mesh,
        scratch_types=[
            pltpu.VMEM((N, CHUNK), jnp.int32),
            pltpu.VMEM((NBUF, CHUNK, D), jnp.float32),
            pltpu.SemaphoreType.DMA((NBUF,)),
            pltpu.SemaphoreType.DMA((NBUF,)),
        ],
    )(table, idx)
    return out_t.transpose(1, 0, 2)


# speedup vs baseline: 13.4328x; 1.0254x over previous
import jax
import jax.numpy as jnp
from jax import lax
from jax.experimental import pallas as pl
from jax.experimental.pallas import tpu as pltpu
from jax.experimental.pallas import tpu_sc as plsc

VOCAB = 100000
D = 128
B = 4096
N = 50

NC = 2
NS = 16
NW = NC * NS

CHUNK = B // NW
NBUF = 7


def _body(table_hbm, idx_hbm, out_hbm, idx_v, rows_v, gsem, wsem):
    wid = lax.axis_index("s") * NC + lax.axis_index("c")
    b0 = wid * CHUNK
    pltpu.sync_copy(idx_hbm.at[:, pl.ds(b0, CHUNK)], idx_v)

    def gather(n, b):
        pltpu.async_copy(table_hbm.at[idx_v.at[n]], rows_v.at[b], gsem.at[b])

    def wait_gather(b):
        pltpu.make_async_copy(
            table_hbm.at[idx_v.at[0]], rows_v.at[b], gsem.at[b]).wait()

    def write(n, b):
        pltpu.async_copy(
            rows_v.at[b], out_hbm.at[n, pl.ds(b0, CHUNK)], wsem.at[b])

    def wait_write(b):
        pltpu.make_async_copy(
            rows_v.at[b], out_hbm.at[0, pl.ds(b0, CHUNK)], wsem.at[b]).wait()

    for b in range(NBUF):
        gather(b, b)
    for n in range(N):
        b = n % NBUF
        wait_gather(b)
        write(n, b)
        if n + NBUF < N:
            wait_write(b)
            gather(n + NBUF, b)
    for n in range(N - NBUF, N):
        wait_write(n % NBUF)


@jax.jit
def kernel(x, table):
    idx = x.astype(jnp.int32).T
    mesh = plsc.VectorSubcoreMesh(core_axis_name="c", subcore_axis_name="s")
    out_t = pl.kernel(
        _body,
        out_type=jax.ShapeDtypeStruct((N, B, D), jnp.float32),
        mesh=mesh,
        scratch_types=[
            pltpu.VMEM((N, CHUNK), jnp.int32),
            pltpu.VMEM((NBUF, CHUNK, D), jnp.float32),
            pltpu.SemaphoreType.DMA((NBUF,)),
            pltpu.SemaphoreType.DMA((NBUF,)),
        ],
    )(table, idx)
    return out_t.transpose(1, 0, 2)


# speedup vs baseline: 13.5296x; 1.0072x over previous
import jax
import jax.numpy as jnp
from jax import lax
from jax.experimental import pallas as pl
from jax.experimental.pallas import tpu as pltpu
from jax.experimental.pallas import tpu_sc as plsc

VOCAB = 100000
D = 128
B = 4096
N = 50

NC = 2
NS = 16
NW = NC * NS

CHUNK = B // NW
NBUF = 7


def _body(table_hbm, idx_hbm, out_hbm, idx_v, rows_v, gsem, wsem):
    wid = lax.axis_index("s") * NC + lax.axis_index("c")
    b0 = wid * CHUNK
    pltpu.sync_copy(idx_hbm.at[pl.ds(0, 8), pl.ds(b0, CHUNK)],
                    idx_v.at[pl.ds(0, 8)])

    def gather(n, b):
        pltpu.async_copy(table_hbm.at[idx_v.at[n]], rows_v.at[b], gsem.at[b])

    def wait_gather(b):
        pltpu.make_async_copy(
            table_hbm.at[idx_v.at[0]], rows_v.at[b], gsem.at[b]).wait()

    def write(n, b):
        pltpu.async_copy(
            rows_v.at[b], out_hbm.at[n, pl.ds(b0, CHUNK)], wsem.at[b])

    def wait_write(b):
        pltpu.make_async_copy(
            rows_v.at[b], out_hbm.at[0, pl.ds(b0, CHUNK)], wsem.at[b]).wait()

    for b in range(NBUF):
        gather(b, b)
    pltpu.sync_copy(idx_hbm.at[pl.ds(8, N - 8), pl.ds(b0, CHUNK)],
                    idx_v.at[pl.ds(8, N - 8)])
    for n in range(N):
        b = n % NBUF
        wait_gather(b)
        write(n, b)
        if n + NBUF < N:
            wait_write(b)
            gather(n + NBUF, b)
    for n in range(N - NBUF, N):
        wait_write(n % NBUF)


@jax.jit
def kernel(x, table):
    idx = x.astype(jnp.int32).T
    mesh = plsc.VectorSubcoreMesh(core_axis_name="c", subcore_axis_name="s")
    out_t = pl.kernel(
        _body,
        out_type=jax.ShapeDtypeStruct((N, B, D), jnp.float32),
        mesh=mesh,
        scratch_types=[
            pltpu.VMEM((N, CHUNK), jnp.int32),
            pltpu.VMEM((NBUF, CHUNK, D), jnp.float32),
            pltpu.SemaphoreType.DMA((NBUF,)),
            pltpu.SemaphoreType.DMA((NBUF,)),
        ],
    )(table, idx)
    return out_t.transpose(1, 0, 2)


# speedup vs baseline: 13.6095x; 1.0059x over previous
import jax
import jax.numpy as jnp
from jax import lax
from jax.experimental import pallas as pl
from jax.experimental.pallas import tpu as pltpu
from jax.experimental.pallas import tpu_sc as plsc

VOCAB = 100000
D = 128
B = 4096
N = 50

NC = 2
NS = 16
NW = NC * NS

CHUNK = B // NW
TPC = 2
NCH = N // TPC
NBUF = 3


def _body(table_hbm, idx_hbm, out_hbm, idx_v, rows_v, gsem, wsem):
    wid = lax.axis_index("s") * NC + lax.axis_index("c")
    b0 = wid * CHUNK
    pltpu.sync_copy(idx_hbm.at[pl.ds(0, 8), pl.ds(b0, CHUNK)],
                    idx_v.at[pl.ds(0, 8)])

    def gather(c, b):
        pltpu.async_copy(
            table_hbm.at[idx_v.at[c * TPC]], rows_v.at[b, 0], gsem.at[b])
        pltpu.async_copy(
            table_hbm.at[idx_v.at[c * TPC + 1]], rows_v.at[b, 1], gsem.at[b])

    def wait_gather(b):
        pltpu.make_async_copy(
            table_hbm.at[idx_v.at[0]], rows_v.at[b], gsem.at[b]).wait()

    def write(c, b):
        pltpu.async_copy(
            rows_v.at[b],
            out_hbm.at[pl.ds(c * TPC, TPC), pl.ds(b0, CHUNK)], wsem.at[b])

    def wait_write(b):
        pltpu.make_async_copy(
            rows_v.at[b],
            out_hbm.at[pl.ds(0, TPC), pl.ds(b0, CHUNK)], wsem.at[b]).wait()

    for b in range(NBUF):
        gather(b, b)
    pltpu.sync_copy(idx_hbm.at[pl.ds(8, N - 8), pl.ds(b0, CHUNK)],
                    idx_v.at[pl.ds(8, N - 8)])
    for c in range(NCH):
        b = c % NBUF
        wait_gather(b)
        write(c, b)
        if c + NBUF < NCH:
            wait_write(b)
            gather(c + NBUF, b)
    for c in range(NCH - NBUF, NCH):
        wait_write(c % NBUF)


@jax.jit
def kernel(x, table):
    idx = x.astype(jnp.int32).T
    mesh = plsc.VectorSubcoreMesh(core_axis_name="c", subcore_axis_name="s")
    out_t = pl.kernel(
        _body,
        out_type=jax.ShapeDtypeStruct((N, B, D), jnp.float32),
        mesh=mesh,
        scratch_types=[
            pltpu.VMEM((N, CHUNK), jnp.int32),
            pltpu.VMEM((NBUF, TPC, CHUNK, D), jnp.float32),
            pltpu.SemaphoreType.DMA((NBUF,)),
            pltpu.SemaphoreType.DMA((NBUF,)),
        ],
    )(table, idx)
    return out_t.transpose(1, 0, 2)
